# baseline (speedup 1.0000x reference)
import jax
import jax.numpy as jnp
from jax import lax
from jax.experimental import pallas as pl
from jax.experimental.pallas import tpu as pltpu

ROWS = 16384
COLS = 1000
BLK_R = 256
GRID = ROWS // BLK_R


def _body(lab_ref, x_ref, out_ref):
    lab_col = lab_ref[...]
    col = lax.broadcasted_iota(jnp.int32, (BLK_R, COLS), 1)
    v = jnp.sum(
        jnp.where(col == lab_col, x_ref[...], 0.0), axis=1, keepdims=True
    )
    out_ref[...] = jnp.exp(jnp.abs(v) * 0.1) - 1.0


@jax.jit
def kernel(x, labels):
    labs2 = labels.astype(jnp.int32).reshape(ROWS, 1)
    return pl.pallas_call(
        _body,
        grid=(GRID,),
        in_specs=[
            pl.BlockSpec((BLK_R, 1), lambda g: (g, 0)),
            pl.BlockSpec((BLK_R, COLS), lambda g: (g, 0)),
        ],
        out_specs=pl.BlockSpec((BLK_R, 1), lambda g: (g, 0)),
        out_shape=jax.ShapeDtypeStruct((ROWS, 1), jnp.float32),
        compiler_params=pltpu.CompilerParams(
            dimension_semantics=("arbitrary",)
        ),
    )(labs2, x)
